# initial kernel scaffold (unmeasured)
import jax
import jax.numpy as jnp
from jax import lax
from jax.experimental import pallas as pl
from jax.experimental.pallas import tpu as pltpu

N_DEV = 4


def kernel(x, Wg, Wu, Wd):
    m, _ = x.shape
    _, d = Wd.shape

    def body(x_ref, wg_ref, wu_ref, wd_ref, out_ref, comm_ref, send_sems, recv_sems):
        my = lax.axis_index("i")
        left = lax.rem(my + N_DEV - 1, N_DEV)
        right = lax.rem(my + 1, N_DEV)

        barrier_sem = pltpu.get_barrier_semaphore()
        for nbr in (left, right):
            pl.semaphore_signal(
                barrier_sem, inc=1,
                device_id=(nbr,), device_id_type=pl.DeviceIdType.MESH,
            )
        pl.semaphore_wait(barrier_sem, 2)

        xb = x_ref[...].astype(jnp.bfloat16)
        gate = jnp.dot(xb, wg_ref[...].astype(jnp.bfloat16),
                       preferred_element_type=jnp.float32)
        up = jnp.dot(xb, wu_ref[...].astype(jnp.bfloat16),
                     preferred_element_type=jnp.float32)
        hidden = (gate * (up * jax.nn.sigmoid(up))).astype(jnp.bfloat16)
        partial = jnp.dot(hidden, wd_ref[...].astype(jnp.bfloat16),
                          preferred_element_type=jnp.float32)

        out_ref[...] = partial
        comm_ref[0, :, :] = partial.astype(jnp.bfloat16)

        for hop in range(N_DEV - 1):
            rdma = pltpu.make_async_remote_copy(
                src_ref=comm_ref.at[hop],
                dst_ref=comm_ref.at[hop + 1],
                send_sem=send_sems.at[hop],
                recv_sem=recv_sems.at[hop],
                device_id=(right,),
                device_id_type=pl.DeviceIdType.MESH,
            )
            rdma.start()
            rdma.wait()
            out_ref[...] = out_ref[...] + comm_ref[hop + 1, :, :].astype(jnp.float32)

    return pl.pallas_call(
        body,
        out_shape=jax.ShapeDtypeStruct((m, d), jnp.float32),
        in_specs=[pl.BlockSpec(memory_space=pltpu.VMEM)] * 4,
        out_specs=pl.BlockSpec(memory_space=pltpu.VMEM),
        scratch_shapes=[
            pltpu.VMEM((N_DEV, m, d), jnp.bfloat16),
            pltpu.SemaphoreType.DMA((N_DEV - 1,)),
            pltpu.SemaphoreType.DMA((N_DEV - 1,)),
        ],
        compiler_params=pltpu.CompilerParams(collective_id=0),
    )(x, Wg, Wu, Wd)


# baseline (device time: 107288 ns/iter reference)
import jax
import jax.numpy as jnp
from jax import lax
from jax.experimental import pallas as pl
from jax.experimental.pallas import tpu as pltpu

N_DEV = 4


def kernel(x, Wg, Wu, Wd):
    m, _ = x.shape
    _, d = Wd.shape

    def body(x_ref, wg_ref, wu_ref, wd_ref, out_ref, comm_ref, send_sems, recv_sems):
        my = lax.axis_index("i")
        left = lax.rem(my + N_DEV - 1, N_DEV)
        right = lax.rem(my + 1, N_DEV)

        barrier_sem = pltpu.get_barrier_semaphore()
        for nbr in (left, right):
            pl.semaphore_signal(
                barrier_sem, inc=1,
                device_id=(nbr,), device_id_type=pl.DeviceIdType.MESH,
            )
        pl.semaphore_wait(barrier_sem, 2)

        xb = x_ref[...].astype(jnp.bfloat16)
        gate = jnp.dot(xb, wg_ref[...].astype(jnp.bfloat16),
                       preferred_element_type=jnp.float32)
        up = jnp.dot(xb, wu_ref[...].astype(jnp.bfloat16),
                     preferred_element_type=jnp.float32)
        hidden = (gate * (up * jax.nn.sigmoid(up))).astype(jnp.bfloat16)
        partial = jnp.dot(hidden, wd_ref[...].astype(jnp.bfloat16),
                          preferred_element_type=jnp.float32)

        out_ref[...] = partial
        comm_ref[0, :, :] = partial.astype(jnp.bfloat16)

        for hop in range(N_DEV - 1):
            rdma = pltpu.make_async_remote_copy(
                src_ref=comm_ref.at[hop],
                dst_ref=comm_ref.at[hop + 1],
                send_sem=send_sems.at[hop],
                recv_sem=recv_sems.at[hop],
                device_id=(right,),
                device_id_type=pl.DeviceIdType.MESH,
            )
            rdma.start()
            rdma.wait()
            out_ref[...] = out_ref[...] + comm_ref[hop + 1, :, :].astype(jnp.float32)

    return pl.pallas_call(
        body,
        out_shape=jax.ShapeDtypeStruct((m, d), jnp.float32),
        in_specs=[pl.BlockSpec(memory_space=pltpu.VMEM)] * 4,
        out_specs=pl.BlockSpec(memory_space=pltpu.VMEM),
        scratch_shapes=[
            pltpu.VMEM((N_DEV, m, d), jnp.bfloat16),
            pltpu.SemaphoreType.DMA((N_DEV - 1,)),
            pltpu.SemaphoreType.DMA((N_DEV - 1,)),
        ],
        compiler_params=pltpu.CompilerParams(
            collective_id=0, vmem_limit_bytes=100 * 1024 * 1024
        ),
    )(x, Wg, Wu, Wd)


# device time: 64135 ns/iter; 1.6728x vs baseline; 1.6728x over previous
import jax
import jax.numpy as jnp
from jax import lax
from jax.experimental import pallas as pl
from jax.experimental.pallas import tpu as pltpu

N_DEV = 4


def kernel(x, Wg, Wu, Wd):
    m, _ = x.shape
    _, d = Wd.shape
    q = m // N_DEV

    def body(x_ref, wg_ref, wu_ref, wd_ref, out_ref,
             rs_send, rs_recv, ag_own, agr_buf, agl_buf,
             rs_ssem, rs_rsem, agr_ssem, agr_rsem, agl_ssem, agl_rsem):
        my = lax.axis_index("i")
        left = lax.rem(my + N_DEV - 1, N_DEV)
        right = lax.rem(my + 1, N_DEV)

        barrier_sem = pltpu.get_barrier_semaphore()
        for nbr in (left, right):
            pl.semaphore_signal(
                barrier_sem, inc=1,
                device_id=(nbr,), device_id_type=pl.DeviceIdType.MESH,
            )
        pl.semaphore_wait(barrier_sem, 2)

        wg_b = wg_ref[...].astype(jnp.bfloat16)
        wu_b = wu_ref[...].astype(jnp.bfloat16)
        wd_b = wd_ref[...].astype(jnp.bfloat16)

        def quarter_partial(c):
            xq = x_ref[pl.ds(c * q, q), :].astype(jnp.bfloat16)
            gate = jnp.dot(xq, wg_b, preferred_element_type=jnp.float32)
            up = jnp.dot(xq, wu_b, preferred_element_type=jnp.float32)
            hidden = (gate * (up * jax.nn.sigmoid(up))).astype(jnp.bfloat16)
            return jnp.dot(hidden, wd_b, preferred_element_type=jnp.float32)

        def rdma(src, dst, ssem, rsem, dev):
            return pltpu.make_async_remote_copy(
                src_ref=src, dst_ref=dst, send_sem=ssem, recv_sem=rsem,
                device_id=(dev,), device_id_type=pl.DeviceIdType.MESH,
            )

        rs = []
        p = quarter_partial(my)
        rs_send[0, :, :] = p.astype(jnp.bfloat16)
        r = rdma(rs_send.at[0], rs_recv.at[0], rs_ssem.at[0], rs_rsem.at[0], right)
        r.start()
        rs.append(r)
        for s in (1, 2):
            c = lax.rem(my - s + N_DEV, N_DEV)
            p = quarter_partial(c)
            rs[s - 1].wait_recv()
            tot = p + rs_recv[s - 1, :, :].astype(jnp.float32)
            rs_send[s, :, :] = tot.astype(jnp.bfloat16)
            r = rdma(rs_send.at[s], rs_recv.at[s], rs_ssem.at[s], rs_rsem.at[s], right)
            r.start()
            rs.append(r)
        c_red = lax.rem(my + 1, N_DEV)
        p = quarter_partial(c_red)
        rs[2].wait_recv()
        red = p + rs_recv[2, :, :].astype(jnp.float32)
        out_ref[pl.ds(c_red * q, q), :] = red

        ag_own[:, :] = red.astype(jnp.bfloat16)
        ag0r = rdma(ag_own, agr_buf.at[0], agr_ssem.at[0], agr_rsem.at[0], right)
        ag0l = rdma(ag_own, agl_buf.at[0], agl_ssem.at[0], agl_rsem.at[0], left)
        ag0r.start()
        ag0l.start()

        ag0r.wait_recv()
        out_ref[pl.ds(my * q, q), :] = agr_buf[0, :, :].astype(jnp.float32)
        ag1r = rdma(agr_buf.at[0], agr_buf.at[1], agr_ssem.at[1], agr_rsem.at[1], right)
        ag1r.start()

        ag0l.wait_recv()
        c2 = lax.rem(my + 2, N_DEV)
        out_ref[pl.ds(c2 * q, q), :] = agl_buf[0, :, :].astype(jnp.float32)

        ag1r.wait_recv()
        out_ref[pl.ds(left * q, q), :] = agr_buf[1, :, :].astype(jnp.float32)

        for r in rs + [ag0r, ag0l, ag1r]:
            r.wait_send()

    return pl.pallas_call(
        body,
        out_shape=jax.ShapeDtypeStruct((m, d), jnp.float32),
        in_specs=[pl.BlockSpec(memory_space=pltpu.VMEM)] * 4,
        out_specs=pl.BlockSpec(memory_space=pltpu.VMEM),
        scratch_shapes=[
            pltpu.VMEM((3, q, d), jnp.bfloat16),
            pltpu.VMEM((3, q, d), jnp.bfloat16),
            pltpu.VMEM((q, d), jnp.bfloat16),
            pltpu.VMEM((2, q, d), jnp.bfloat16),
            pltpu.VMEM((1, q, d), jnp.bfloat16),
            pltpu.SemaphoreType.DMA((3,)),
            pltpu.SemaphoreType.DMA((3,)),
            pltpu.SemaphoreType.DMA((2,)),
            pltpu.SemaphoreType.DMA((2,)),
            pltpu.SemaphoreType.DMA((1,)),
            pltpu.SemaphoreType.DMA((1,)),
        ],
        compiler_params=pltpu.CompilerParams(
            collective_id=0, vmem_limit_bytes=100 * 1024 * 1024
        ),
    )(x, Wg, Wu, Wd)


# device time: 61345 ns/iter; 1.7489x vs baseline; 1.0455x over previous
import jax
import jax.numpy as jnp
from jax import lax
from jax.experimental import pallas as pl
from jax.experimental.pallas import tpu as pltpu

N_DEV = 4


def kernel(x, Wg, Wu, Wd):
    m, _ = x.shape
    _, d = Wd.shape
    q = m // N_DEV

    def body(x_ref, wg_ref, wu_ref, wd_ref, out_ref,
             rs_send, rs_recv, ag_own, agr_buf, agl_buf, agf_buf,
             rs_ssem, rs_rsem, agr_ssem, agr_rsem, agl_ssem, agl_rsem):
        my = lax.axis_index("i")
        left = lax.rem(my + N_DEV - 1, N_DEV)
        right = lax.rem(my + 1, N_DEV)

        barrier_sem = pltpu.get_barrier_semaphore()
        for nbr in (left, right):
            pl.semaphore_signal(
                barrier_sem, inc=1,
                device_id=(nbr,), device_id_type=pl.DeviceIdType.MESH,
            )
        pl.semaphore_wait(barrier_sem, 2)

        wg_b = wg_ref[...].astype(jnp.bfloat16)
        wu_b = wu_ref[...].astype(jnp.bfloat16)
        wd_b = wd_ref[...].astype(jnp.bfloat16)

        def quarter_partial(c):
            xq = x_ref[pl.ds(c * q, q), :].astype(jnp.bfloat16)
            gate = jnp.dot(xq, wg_b, preferred_element_type=jnp.float32)
            up = jnp.dot(xq, wu_b, preferred_element_type=jnp.float32)
            hidden = (gate * (up * jax.nn.sigmoid(up))).astype(jnp.bfloat16)
            return jnp.dot(hidden, wd_b, preferred_element_type=jnp.float32)

        def rdma(src, dst, ssem, rsem, dev):
            return pltpu.make_async_remote_copy(
                src_ref=src, dst_ref=dst, send_sem=ssem, recv_sem=rsem,
                device_id=(dev,), device_id_type=pl.DeviceIdType.MESH,
            )

        rs = []
        p = quarter_partial(my)
        rs_send[0, :, :] = p.astype(jnp.bfloat16)
        r = rdma(rs_send.at[0], rs_recv.at[0], rs_ssem.at[0], rs_rsem.at[0], right)
        r.start()
        rs.append(r)
        for s in (1, 2):
            c = lax.rem(my - s + N_DEV, N_DEV)
            p = quarter_partial(c)
            rs[s - 1].wait_recv()
            tot = p + rs_recv[s - 1, :, :].astype(jnp.float32)
            rs_send[s, :, :] = tot.astype(jnp.bfloat16)
            r = rdma(rs_send.at[s], rs_recv.at[s], rs_ssem.at[s], rs_rsem.at[s], right)
            r.start()
            rs.append(r)
        c_red = lax.rem(my + 1, N_DEV)
        p = quarter_partial(c_red)
        rs[2].wait_recv()
        red = p + rs_recv[2, :, :].astype(jnp.float32)

        ag_own[:, :] = red.astype(jnp.bfloat16)
        ag0r = rdma(ag_own, agr_buf, agr_ssem.at[0], agr_rsem.at[0], right)
        ag0l = rdma(ag_own, agl_buf, agl_ssem.at[0], agl_rsem.at[0], left)
        ag0r.start()
        ag0l.start()
        out_ref[pl.ds(c_red * q, q), :] = red

        ag0r.wait_recv()
        ag1r = rdma(agr_buf.at[:, pl.ds(0, d // 2)],
                    agf_buf.at[:, pl.ds(0, d // 2)],
                    agr_ssem.at[1], agr_rsem.at[1], right)
        ag1r.start()
        out_ref[pl.ds(my * q, q), :] = agr_buf[:, :].astype(jnp.float32)

        ag0l.wait_recv()
        ag1l = rdma(agl_buf.at[:, pl.ds(d // 2, d // 2)],
                    agf_buf.at[:, pl.ds(d // 2, d // 2)],
                    agl_ssem.at[1], agl_rsem.at[1], left)
        ag1l.start()
        c2 = lax.rem(my + 2, N_DEV)
        out_ref[pl.ds(c2 * q, q), :] = agl_buf[:, :].astype(jnp.float32)

        ag1r.wait_recv()
        ag1l.wait_recv()
        out_ref[pl.ds(left * q, q), :] = agf_buf[:, :].astype(jnp.float32)

        for r in rs + [ag0r, ag0l, ag1r, ag1l]:
            r.wait_send()

    return pl.pallas_call(
        body,
        out_shape=jax.ShapeDtypeStruct((m, d), jnp.float32),
        in_specs=[pl.BlockSpec(memory_space=pltpu.VMEM)] * 4,
        out_specs=pl.BlockSpec(memory_space=pltpu.VMEM),
        scratch_shapes=[
            pltpu.VMEM((3, q, d), jnp.bfloat16),
            pltpu.VMEM((3, q, d), jnp.bfloat16),
            pltpu.VMEM((q, d), jnp.bfloat16),
            pltpu.VMEM((q, d), jnp.bfloat16),
            pltpu.VMEM((q, d), jnp.bfloat16),
            pltpu.VMEM((q, d), jnp.bfloat16),
            pltpu.SemaphoreType.DMA((3,)),
            pltpu.SemaphoreType.DMA((3,)),
            pltpu.SemaphoreType.DMA((2,)),
            pltpu.SemaphoreType.DMA((2,)),
            pltpu.SemaphoreType.DMA((2,)),
            pltpu.SemaphoreType.DMA((2,)),
        ],
        compiler_params=pltpu.CompilerParams(
            collective_id=0, vmem_limit_bytes=100 * 1024 * 1024
        ),
    )(x, Wg, Wu, Wd)


# device time: 54046 ns/iter; 1.9851x vs baseline; 1.1351x over previous
import jax
import jax.numpy as jnp
from jax import lax
from jax.experimental import pallas as pl
from jax.experimental.pallas import tpu as pltpu

N_DEV = 4


def kernel(x, Wg, Wu, Wd):
    m, _ = x.shape
    _, d = Wd.shape
    q = m // N_DEV

    def body(x_ref, wg_ref, wu_ref, wd_ref, out_ref,
             a2a_send, a2a_recv, ag_own, agr_buf, agl_buf, agf_buf,
             a2a_ssem, a2a_rsem, agr_ssem, agr_rsem, agl_ssem, agl_rsem):
        my = lax.axis_index("i")
        left = lax.rem(my + N_DEV - 1, N_DEV)
        right = lax.rem(my + 1, N_DEV)
        diag = lax.rem(my + 2, N_DEV)

        barrier_sem = pltpu.get_barrier_semaphore()
        for nbr in (left, right, diag):
            pl.semaphore_signal(
                barrier_sem, inc=1,
                device_id=(nbr,), device_id_type=pl.DeviceIdType.MESH,
            )
        pl.semaphore_wait(barrier_sem, 3)

        wg_b = wg_ref[...].astype(jnp.bfloat16)
        wu_b = wu_ref[...].astype(jnp.bfloat16)
        wd_b = wd_ref[...].astype(jnp.bfloat16)

        def quarter_partial(c):
            xq = x_ref[pl.ds(c * q, q), :].astype(jnp.bfloat16)
            gate = jnp.dot(xq, wg_b, preferred_element_type=jnp.float32)
            up = jnp.dot(xq, wu_b, preferred_element_type=jnp.float32)
            hidden = (gate * (up * jax.nn.sigmoid(up))).astype(jnp.bfloat16)
            return jnp.dot(hidden, wd_b, preferred_element_type=jnp.float32)

        def rdma(src, dst, ssem, rsem, dev):
            return pltpu.make_async_remote_copy(
                src_ref=src, dst_ref=dst, send_sem=ssem, recv_sem=rsem,
                device_id=(dev,), device_id_type=pl.DeviceIdType.MESH,
            )

        a2a = []
        for k in (1, 2, 3):
            c = lax.rem(my + k, N_DEV)
            p = quarter_partial(c)
            a2a_send[k - 1, :, :] = p.astype(jnp.bfloat16)
            r = rdma(a2a_send.at[k - 1], a2a_recv.at[k - 1],
                     a2a_ssem.at[k - 1], a2a_rsem.at[k - 1], c)
            r.start()
            a2a.append(r)
        p_own = quarter_partial(my)
        for r in a2a:
            r.wait_recv()
        red = p_own + (a2a_recv[0, :, :].astype(jnp.float32)
                       + a2a_recv[1, :, :].astype(jnp.float32)
                       + a2a_recv[2, :, :].astype(jnp.float32))

        ag_own[:, :] = red.astype(jnp.bfloat16)
        ag0r = rdma(ag_own, agr_buf, agr_ssem.at[0], agr_rsem.at[0], right)
        ag0l = rdma(ag_own, agl_buf, agl_ssem.at[0], agl_rsem.at[0], left)
        ag0r.start()
        ag0l.start()
        out_ref[pl.ds(my * q, q), :] = red

        ag0r.wait_recv()
        ag1r = rdma(agr_buf.at[:, pl.ds(0, d // 2)],
                    agf_buf.at[:, pl.ds(0, d // 2)],
                    agr_ssem.at[1], agr_rsem.at[1], right)
        ag1r.start()
        out_ref[pl.ds(left * q, q), :] = agr_buf[:, :].astype(jnp.float32)

        ag0l.wait_recv()
        ag1l = rdma(agl_buf.at[:, pl.ds(d // 2, d // 2)],
                    agf_buf.at[:, pl.ds(d // 2, d // 2)],
                    agl_ssem.at[1], agl_rsem.at[1], left)
        ag1l.start()
        out_ref[pl.ds(right * q, q), :] = agl_buf[:, :].astype(jnp.float32)

        ag1r.wait_recv()
        ag1l.wait_recv()
        out_ref[pl.ds(diag * q, q), :] = agf_buf[:, :].astype(jnp.float32)

        for r in a2a + [ag0r, ag0l, ag1r, ag1l]:
            r.wait_send()

    return pl.pallas_call(
        body,
        out_shape=jax.ShapeDtypeStruct((m, d), jnp.float32),
        in_specs=[pl.BlockSpec(memory_space=pltpu.VMEM)] * 4,
        out_specs=pl.BlockSpec(memory_space=pltpu.VMEM),
        scratch_shapes=[
            pltpu.VMEM((3, q, d), jnp.bfloat16),
            pltpu.VMEM((3, q, d), jnp.bfloat16),
            pltpu.VMEM((q, d), jnp.bfloat16),
            pltpu.VMEM((q, d), jnp.bfloat16),
            pltpu.VMEM((q, d), jnp.bfloat16),
            pltpu.VMEM((q, d), jnp.bfloat16),
            pltpu.SemaphoreType.DMA((3,)),
            pltpu.SemaphoreType.DMA((3,)),
            pltpu.SemaphoreType.DMA((2,)),
            pltpu.SemaphoreType.DMA((2,)),
            pltpu.SemaphoreType.DMA((2,)),
            pltpu.SemaphoreType.DMA((2,)),
        ],
        compiler_params=pltpu.CompilerParams(
            collective_id=0, vmem_limit_bytes=100 * 1024 * 1024
        ),
    )(x, Wg, Wu, Wd)


# device time: 45908 ns/iter; 2.3370x vs baseline; 1.1773x over previous
import jax
import jax.numpy as jnp
from jax import lax
from jax.experimental import pallas as pl
from jax.experimental.pallas import tpu as pltpu

N_DEV = 4


def kernel(x, Wg, Wu, Wd):
    m, _ = x.shape
    _, d = Wd.shape
    q = m // N_DEV

    def body(x_ref, wg_ref, wu_hbm, wd_hbm, out_ref,
             wu_vmem, wd_vmem,
             a2a_send, a2a_recv, ag_own, agr_buf, agl_buf, agf_buf,
             cp_sem_u, cp_sem_d,
             a2a_ssem, a2a_rsem, agr_ssem, agr_rsem, agl_ssem, agl_rsem):
        my = lax.axis_index("i")
        left = lax.rem(my + N_DEV - 1, N_DEV)
        right = lax.rem(my + 1, N_DEV)
        diag = lax.rem(my + 2, N_DEV)

        cp_wu = pltpu.make_async_copy(wu_hbm, wu_vmem, cp_sem_u)
        cp_wd = pltpu.make_async_copy(wd_hbm, wd_vmem, cp_sem_d)
        cp_wu.start()
        cp_wd.start()

        barrier_sem = pltpu.get_barrier_semaphore()
        for nbr in (left, right, diag):
            pl.semaphore_signal(
                barrier_sem, inc=1,
                device_id=(nbr,), device_id_type=pl.DeviceIdType.MESH,
            )

        wg_b = wg_ref[...].astype(jnp.bfloat16)
        xq0 = x_ref[pl.ds(diag * q, q), :].astype(jnp.bfloat16)
        gate0 = jnp.dot(xq0, wg_b, preferred_element_type=jnp.float32)
        cp_wu.wait()
        wu_b = wu_vmem[...].astype(jnp.bfloat16)
        up0 = jnp.dot(xq0, wu_b, preferred_element_type=jnp.float32)
        hid_d = (gate0 * (up0 * jax.nn.sigmoid(up0))).astype(jnp.bfloat16)
        cp_wd.wait()
        wd_b = wd_vmem[...].astype(jnp.bfloat16)

        def quarter_hidden(c):
            xq = x_ref[pl.ds(c * q, q), :].astype(jnp.bfloat16)
            gate = jnp.dot(xq, wg_b, preferred_element_type=jnp.float32)
            up = jnp.dot(xq, wu_b, preferred_element_type=jnp.float32)
            return (gate * (up * jax.nn.sigmoid(up))).astype(jnp.bfloat16)

        def rdma(src, dst, ssem, rsem, dev):
            return pltpu.make_async_remote_copy(
                src_ref=src, dst_ref=dst, send_sem=ssem, recv_sem=rsem,
                device_id=(dev,), device_id_type=pl.DeviceIdType.MESH,
            )

        h = d // 2
        wd_lo = wd_b[:, :h]
        wd_hi = wd_b[:, h:]

        def half_send(hid, buf_idx, col0, wd_half, sem_idx, dev):
            cols = pl.ds(col0, h)
            a2a_send[buf_idx, :, cols] = jnp.dot(
                hid, wd_half, preferred_element_type=jnp.float32
            ).astype(jnp.bfloat16)
            r = rdma(a2a_send.at[buf_idx, :, cols], a2a_recv.at[buf_idx, :, cols],
                     a2a_ssem.at[sem_idx], a2a_rsem.at[sem_idx], dev)
            r.start()
            return r

        pl.semaphore_wait(barrier_sem, 3)
        r_da = half_send(hid_d, 0, 0, wd_lo, 0, diag)
        hid_l = quarter_hidden(left)
        r_la = half_send(hid_l, 2, 0, wd_lo, 1, left)
        r_db = half_send(hid_d, 0, h, wd_hi, 2, diag)
        hid_r = quarter_hidden(right)
        r_ra = half_send(hid_r, 1, 0, wd_lo, 3, right)
        r_lb = half_send(hid_l, 2, h, wd_hi, 4, left)
        r_rb = half_send(hid_r, 1, h, wd_hi, 5, right)
        a2a = [r_da, r_la, r_db, r_ra, r_lb, r_rb]

        hid_o = quarter_hidden(my)
        p_own = jnp.dot(hid_o, wd_b, preferred_element_type=jnp.float32)
        r_da.wait_recv()
        r_db.wait_recv()
        acc = p_own + a2a_recv[0, :, :].astype(jnp.float32)
        r_la.wait_recv()
        r_lb.wait_recv()
        acc = acc + a2a_recv[2, :, :].astype(jnp.float32)
        r_ra.wait_recv()
        r_rb.wait_recv()
        red = acc + a2a_recv[1, :, :].astype(jnp.float32)

        lo = pl.ds(0, h)
        hi = pl.ds(h, h)
        ag_own[:, :] = red.astype(jnp.bfloat16)
        r0r_lo = rdma(ag_own.at[:, lo], agr_buf.at[:, lo],
                      agr_ssem.at[0], agr_rsem.at[0], right)
        r0l_hi = rdma(ag_own.at[:, hi], agl_buf.at[:, hi],
                      agl_ssem.at[0], agl_rsem.at[0], left)
        r0r_hi = rdma(ag_own.at[:, hi], agr_buf.at[:, hi],
                      agr_ssem.at[1], agr_rsem.at[1], right)
        r0l_lo = rdma(ag_own.at[:, lo], agl_buf.at[:, lo],
                      agl_ssem.at[1], agl_rsem.at[1], left)
        r0r_lo.start()
        r0l_hi.start()
        r0r_hi.start()
        r0l_lo.start()
        out_ref[pl.ds(my * q, q), :] = red

        r0r_lo.wait_recv()
        fwd_r = rdma(agr_buf.at[:, lo], agf_buf.at[:, lo],
                     agr_ssem.at[2], agr_rsem.at[2], right)
        fwd_r.start()
        r0l_hi.wait_recv()
        fwd_l = rdma(agl_buf.at[:, hi], agf_buf.at[:, hi],
                     agl_ssem.at[2], agl_rsem.at[2], left)
        fwd_l.start()

        r0r_hi.wait_recv()
        out_ref[pl.ds(left * q, q), :] = agr_buf[:, :].astype(jnp.float32)
        r0l_lo.wait_recv()
        out_ref[pl.ds(right * q, q), :] = agl_buf[:, :].astype(jnp.float32)
        fwd_r.wait_recv()
        fwd_l.wait_recv()
        out_ref[pl.ds(diag * q, q), :] = agf_buf[:, :].astype(jnp.float32)

        for r in a2a + [r0r_lo, r0r_hi, r0l_lo, r0l_hi, fwd_r, fwd_l]:
            r.wait_send()

    return pl.pallas_call(
        body,
        out_shape=jax.ShapeDtypeStruct((m, d), jnp.float32),
        in_specs=[
            pl.BlockSpec(memory_space=pltpu.VMEM),
            pl.BlockSpec(memory_space=pltpu.VMEM),
            pl.BlockSpec(memory_space=pl.ANY),
            pl.BlockSpec(memory_space=pl.ANY),
        ],
        out_specs=pl.BlockSpec(memory_space=pltpu.VMEM),
        scratch_shapes=[
            pltpu.VMEM(Wu.shape, jnp.float32),
            pltpu.VMEM(Wd.shape, jnp.float32),
            pltpu.VMEM((3, q, d), jnp.bfloat16),
            pltpu.VMEM((3, q, d), jnp.bfloat16),
            pltpu.VMEM((q, d), jnp.bfloat16),
            pltpu.VMEM((q, d), jnp.bfloat16),
            pltpu.VMEM((q, d), jnp.bfloat16),
            pltpu.VMEM((q, d), jnp.bfloat16),
            pltpu.SemaphoreType.DMA,
            pltpu.SemaphoreType.DMA,
            pltpu.SemaphoreType.DMA((6,)),
            pltpu.SemaphoreType.DMA((6,)),
            pltpu.SemaphoreType.DMA((3,)),
            pltpu.SemaphoreType.DMA((3,)),
            pltpu.SemaphoreType.DMA((3,)),
            pltpu.SemaphoreType.DMA((3,)),
        ],
        compiler_params=pltpu.CompilerParams(
            collective_id=0, vmem_limit_bytes=100 * 1024 * 1024
        ),
    )(x, Wg, Wu, Wd)
